# baseline (speedup 1.0000x reference)
import functools

import jax
import jax.numpy as jnp
from jax import lax
from jax.experimental import pallas as pl
from jax.experimental.pallas import tpu as pltpu
from jax.experimental.pallas import tpu_sc as plsc

EMB = 64
SCALE = 8.0

NC = 2
NS = 16
NW = NC * NS

BW = 128


def _sc_embed(gidx2d, pb2d, tab128):
    t_dim, b_dim = gidx2d.shape
    nbh = b_dim // BW
    nblk = t_dim * nbh
    blk_per_w = nblk // NW

    mesh = plsc.VectorSubcoreMesh(core_axis_name="c", subcore_axis_name="s")

    @functools.partial(
        pl.kernel,
        mesh=mesh,
        out_type=jax.ShapeDtypeStruct((t_dim, 8, nbh, 8, BW), jnp.float32),
        scratch_types=[
            pltpu.VMEM((2, BW), jnp.int32),
            pltpu.VMEM((2, BW), jnp.int32),
            pltpu.VMEM((2, BW, BW), jnp.float32),
            pltpu.VMEM((2, 8, 8, BW), jnp.float32),
            pltpu.SemaphoreType.DMA((2,)),
            pltpu.SemaphoreType.DMA((2,)),
            pltpu.SemaphoreType.DMA((2,)),
        ],
        compiler_params=pltpu.CompilerParams(use_tc_tiling_on_sc=False,
                                             needs_layout_passes=False),
    )
    def k(gidx_hbm, pb_hbm, tab_hbm, out_hbm, idx_v, pb_v, rows_v, tb_v,
          isem, gsem, ssem):
        wid = lax.axis_index("s") * NC + lax.axis_index("c")
        f0 = wid * blk_per_w
        iota16 = jnp.arange(16, dtype=jnp.int32)

        def tb_of(f):
            return f // nbh, f % nbh

        def fire_idx(f, b):
            t, bh = tb_of(f)
            pltpu.async_copy(gidx_hbm.at[t, pl.ds(bh * BW, BW)],
                             idx_v.at[b], isem.at[b])
            pltpu.async_copy(pb_hbm.at[t, pl.ds(bh * BW, BW)],
                             pb_v.at[b], isem.at[b])

        def wait_idx(b):
            pltpu.make_async_copy(gidx_hbm.at[0, pl.ds(0, BW)],
                                  idx_v.at[b], isem.at[b]).wait()
            pltpu.make_async_copy(pb_hbm.at[0, pl.ds(0, BW)],
                                  pb_v.at[b], isem.at[b]).wait()

        def fire_gather(b):
            pltpu.async_copy(tab_hbm.at[idx_v.at[b]], rows_v.at[b],
                             gsem.at[b])

        def wait_gather(b):
            pltpu.make_async_copy(tab_hbm.at[pl.ds(0, BW)],
                                  rows_v.at[b], gsem.at[b]).wait()

        def fire_store(f, b):
            t, bh = tb_of(f)
            pltpu.async_copy(tb_v.at[b], out_hbm.at[t, :, bh], ssem.at[b])

        def wait_store(b):
            pltpu.make_async_copy(tb_v.at[b], out_hbm.at[0, :, 0],
                                  ssem.at[b]).wait()

        def transpose_scale(b):
            def body(e, acc):
                ehi = e >> 3
                elo = e & 7
                for kk in range(8):
                    sl = pl.ds(kk * 16, 16)
                    idx0 = iota16 + (kk * 16)
                    idx1 = pb_v[b, sl] + e
                    v = plsc.load_gather(rows_v.at[b], [idx0, idx1])
                    tb_v[b, ehi, elo, sl] = v * SCALE
                return acc

            lax.fori_loop(0, EMB, body, 0)

        fire_idx(f0, 0)
        wait_idx(0)
        fire_gather(0)
        fire_idx(f0 + 1, 1)

        def step(i, carry):
            for b in range(2):
                g = i * 2 + b
                wait_gather(b)
                transpose_scale(b)
                fire_store(f0 + g, b)

                @pl.when(g + 1 < blk_per_w)
                def _():
                    wait_idx(1 - b)

                    @pl.when(g >= 1)
                    def _():
                        wait_store(1 - b)

                    fire_gather(1 - b)

                    @pl.when(g + 2 < blk_per_w)
                    def _():
                        fire_idx(f0 + g + 2, b)

            return carry

        lax.fori_loop(0, blk_per_w // 2, step, 0)
        wait_store(0)
        wait_store(1)

    return k(gidx2d, pb2d, tab128)


def kernel(tokens, table):
    b0, b1 = tokens.shape
    tokT = jnp.swapaxes(tokens, 0, 1).astype(jnp.int32)
    gidx2d = tokT >> 1
    pb2d = (tokT & 1) << 6
    tab128 = jnp.reshape(table, (table.shape[0] // 2, 2 * EMB))
    outv = _sc_embed(gidx2d, pb2d, tab128)
    r = jnp.transpose(outv, (0, 1, 3, 2, 4))
    r = jnp.reshape(r, (b1, EMB, b0))
    return jnp.transpose(r, (2, 0, 1))


# speedup vs baseline: 1.8857x; 1.8857x over previous
import functools

import jax
import jax.numpy as jnp
from jax import lax
from jax.experimental import pallas as pl
from jax.experimental.pallas import tpu as pltpu
from jax.experimental.pallas import tpu_sc as plsc

EMB = 64
SCALE = 8.0

NC = 2
NS = 16
NW = NC * NS

BW = 128


def _sc_embed(gidx2d, pb2d, tab128):
    t_dim, b_dim = gidx2d.shape
    nbh = b_dim // BW
    nblk = t_dim * nbh
    blk_per_w = nblk // NW

    mesh = plsc.VectorSubcoreMesh(core_axis_name="c", subcore_axis_name="s")

    @functools.partial(
        pl.kernel,
        mesh=mesh,
        out_type=jax.ShapeDtypeStruct((t_dim, 8, nbh, 8, BW), jnp.float32),
        scratch_types=[
            pltpu.VMEM((2, BW), jnp.int32),
            pltpu.VMEM((2, BW), jnp.int32),
            pltpu.VMEM((2, BW, BW), jnp.float32),
            pltpu.VMEM((2, 8, 8, BW), jnp.float32),
            pltpu.SemaphoreType.DMA((2,)),
            pltpu.SemaphoreType.DMA((2,)),
            pltpu.SemaphoreType.DMA((2,)),
        ],
        compiler_params=pltpu.CompilerParams(use_tc_tiling_on_sc=False,
                                             needs_layout_passes=False),
    )
    def k(gidx_hbm, pb_hbm, tab_hbm, out_hbm, idx_v, pb_v, rows_v, tb_v,
          isem, gsem, ssem):
        wid = lax.axis_index("s") * NC + lax.axis_index("c")
        f0 = wid * blk_per_w
        iota16 = jnp.arange(16, dtype=jnp.int32)

        def tb_of(f):
            return f // nbh, f % nbh

        def fire_idx(f, b):
            t, bh = tb_of(f)
            pltpu.async_copy(gidx_hbm.at[t, pl.ds(bh * BW, BW)],
                             idx_v.at[b], isem.at[b])
            pltpu.async_copy(pb_hbm.at[t, pl.ds(bh * BW, BW)],
                             pb_v.at[b], isem.at[b])

        def wait_idx(b):
            pltpu.make_async_copy(gidx_hbm.at[0, pl.ds(0, BW)],
                                  idx_v.at[b], isem.at[b]).wait()
            pltpu.make_async_copy(pb_hbm.at[0, pl.ds(0, BW)],
                                  pb_v.at[b], isem.at[b]).wait()

        def fire_gather(b):
            pltpu.async_copy(tab_hbm.at[idx_v.at[b]], rows_v.at[b],
                             gsem.at[b])

        def wait_gather(b):
            pltpu.make_async_copy(tab_hbm.at[pl.ds(0, BW)],
                                  rows_v.at[b], gsem.at[b]).wait()

        def fire_store(f, b):
            t, bh = tb_of(f)
            pltpu.async_copy(tb_v.at[b], out_hbm.at[t, :, bh], ssem.at[b])

        def wait_store(b):
            pltpu.make_async_copy(tb_v.at[b], out_hbm.at[0, :, 0],
                                  ssem.at[b]).wait()

        def transpose_scale(b):
            pbs = [pb_v[b, pl.ds(kk * 16, 16)] for kk in range(8)]
            iotas = [iota16 + (kk * 16) for kk in range(8)]

            @plsc.parallel_loop(0, EMB, step=1, unroll=4)
            def _(e):
                ehi = e >> 3
                elo = e & 7
                for kk in range(8):
                    v = plsc.load_gather(rows_v.at[b],
                                         [iotas[kk], pbs[kk] + e])
                    tb_v[b, ehi, elo, pl.ds(kk * 16, 16)] = v * SCALE

        fire_idx(f0, 0)
        wait_idx(0)
        fire_gather(0)
        fire_idx(f0 + 1, 1)

        def step(i, carry):
            for b in range(2):
                g = i * 2 + b
                wait_gather(b)
                transpose_scale(b)
                fire_store(f0 + g, b)

                @pl.when(g + 1 < blk_per_w)
                def _():
                    wait_idx(1 - b)

                    @pl.when(g >= 1)
                    def _():
                        wait_store(1 - b)

                    fire_gather(1 - b)

                    @pl.when(g + 2 < blk_per_w)
                    def _():
                        fire_idx(f0 + g + 2, b)

            return carry

        lax.fori_loop(0, blk_per_w // 2, step, 0)
        wait_store(0)
        wait_store(1)

    return k(gidx2d, pb2d, tab128)


def kernel(tokens, table):
    b0, b1 = tokens.shape
    tokT = jnp.swapaxes(tokens, 0, 1).astype(jnp.int32)
    gidx2d = tokT >> 1
    pb2d = (tokT & 1) << 6
    tab128 = jnp.reshape(table, (table.shape[0] // 2, 2 * EMB))
    outv = _sc_embed(gidx2d, pb2d, tab128)
    r = jnp.transpose(outv, (0, 1, 3, 2, 4))
    r = jnp.reshape(r, (b1, EMB, b0))
    return jnp.transpose(r, (2, 0, 1))


# speedup vs baseline: 1.8895x; 1.0020x over previous
import functools

import jax
import jax.numpy as jnp
from jax import lax
from jax.experimental import pallas as pl
from jax.experimental.pallas import tpu as pltpu
from jax.experimental.pallas import tpu_sc as plsc

EMB = 64
SCALE = 8.0

NC = 2
NS = 16
NW = NC * NS

BW = 128


def _sc_embed(gidx2d, pb2d, tab128):
    t_dim, b_dim = gidx2d.shape
    nbh = b_dim // BW
    nblk = t_dim * nbh
    blk_per_w = nblk // NW

    mesh = plsc.VectorSubcoreMesh(core_axis_name="c", subcore_axis_name="s")

    @functools.partial(
        pl.kernel,
        mesh=mesh,
        out_type=jax.ShapeDtypeStruct((t_dim, 8, nbh, 8, BW), jnp.float32),
        scratch_types=[
            pltpu.VMEM((2, BW), jnp.int32),
            pltpu.VMEM((2, BW), jnp.int32),
            pltpu.VMEM((2, BW, BW), jnp.float32),
            pltpu.VMEM((2, 8, 8, BW), jnp.float32),
            pltpu.SemaphoreType.DMA((2,)),
            pltpu.SemaphoreType.DMA((2,)),
            pltpu.SemaphoreType.DMA((2,)),
        ],
        compiler_params=pltpu.CompilerParams(use_tc_tiling_on_sc=True,
                                             needs_layout_passes=False),
    )
    def k(gidx_hbm, pb_hbm, tab_hbm, out_hbm, idx_v, pb_v, rows_v, tb_v,
          isem, gsem, ssem):
        wid = lax.axis_index("s") * NC + lax.axis_index("c")
        f0 = wid * blk_per_w
        iota16 = jnp.arange(16, dtype=jnp.int32)

        def tb_of(f):
            return f // nbh, f % nbh

        def fire_idx(f, b):
            t, bh = tb_of(f)
            pltpu.async_copy(gidx_hbm.at[t, pl.ds(bh * BW, BW)],
                             idx_v.at[b], isem.at[b])
            pltpu.async_copy(pb_hbm.at[t, pl.ds(bh * BW, BW)],
                             pb_v.at[b], isem.at[b])

        def wait_idx(b):
            pltpu.make_async_copy(gidx_hbm.at[0, pl.ds(0, BW)],
                                  idx_v.at[b], isem.at[b]).wait()
            pltpu.make_async_copy(pb_hbm.at[0, pl.ds(0, BW)],
                                  pb_v.at[b], isem.at[b]).wait()

        def fire_gather(b):
            pltpu.async_copy(tab_hbm.at[idx_v.at[b]], rows_v.at[b],
                             gsem.at[b])

        def wait_gather(b):
            pltpu.make_async_copy(tab_hbm.at[pl.ds(0, BW)],
                                  rows_v.at[b], gsem.at[b]).wait()

        def fire_store(f, b):
            t, bh = tb_of(f)
            pltpu.async_copy(tb_v.at[b], out_hbm.at[t, :, bh], ssem.at[b])

        def wait_store(b):
            pltpu.make_async_copy(tb_v.at[b], out_hbm.at[0, :, 0],
                                  ssem.at[b]).wait()

        def transpose_scale(b):
            pbs = [pb_v[b, pl.ds(kk * 16, 16)] for kk in range(8)]
            iotas = [iota16 + (kk * 16) for kk in range(8)]

            @plsc.parallel_loop(0, EMB, step=1, unroll=4)
            def _(e):
                ehi = e >> 3
                elo = e & 7
                for kk in range(8):
                    v = plsc.load_gather(rows_v.at[b],
                                         [iotas[kk], pbs[kk] + e])
                    tb_v[b, ehi, elo, pl.ds(kk * 16, 16)] = v * SCALE

        fire_idx(f0, 0)
        wait_idx(0)
        fire_gather(0)
        fire_idx(f0 + 1, 1)

        def step(i, carry):
            for b in range(2):
                g = i * 2 + b
                wait_gather(b)
                transpose_scale(b)
                fire_store(f0 + g, b)

                @pl.when(g + 1 < blk_per_w)
                def _():
                    wait_idx(1 - b)

                    @pl.when(g >= 1)
                    def _():
                        wait_store(1 - b)

                    fire_gather(1 - b)

                    @pl.when(g + 2 < blk_per_w)
                    def _():
                        fire_idx(f0 + g + 2, b)

            return carry

        lax.fori_loop(0, blk_per_w // 2, step, 0)
        wait_store(0)
        wait_store(1)

    return k(gidx2d, pb2d, tab128)


def kernel(tokens, table):
    b0, b1 = tokens.shape
    tokT = jnp.swapaxes(tokens, 0, 1).astype(jnp.int32)
    gidx2d = tokT >> 1
    pb2d = (tokT & 1) << 6
    tab128 = jnp.reshape(table, (table.shape[0] // 2, 2 * EMB))
    outv = _sc_embed(gidx2d, pb2d, tab128)
    r = jnp.transpose(outv, (0, 1, 3, 2, 4))
    r = jnp.reshape(r, (b1, EMB, b0))
    return jnp.transpose(r, (2, 0, 1))


# speedup vs baseline: 2.3220x; 1.2289x over previous
import functools

import jax
import jax.numpy as jnp
from jax import lax
from jax.experimental import pallas as pl
from jax.experimental.pallas import tpu as pltpu
from jax.experimental.pallas import tpu_sc as plsc

EMB = 64
SCALE = 8.0

NC = 2
NS = 16
NW = NC * NS

BW = 128


def _sc_embed(gidx2d, pb2d, tab128):
    t_dim, b_dim = gidx2d.shape
    nbh = b_dim // BW
    nblk = t_dim * nbh
    blk_per_w = nblk // NW

    mesh = plsc.VectorSubcoreMesh(core_axis_name="c", subcore_axis_name="s")

    @functools.partial(
        pl.kernel,
        mesh=mesh,
        out_type=jax.ShapeDtypeStruct((t_dim, 8, nbh, 8, BW), jnp.float32),
        scratch_types=[
            pltpu.VMEM((4, BW), jnp.int32),
            pltpu.VMEM((4, BW), jnp.int32),
            pltpu.VMEM((4, BW, BW), jnp.float32),
            pltpu.VMEM((2, 8, 8, BW), jnp.float32),
            pltpu.SemaphoreType.DMA((4,)),
            pltpu.SemaphoreType.DMA((4,)),
            pltpu.SemaphoreType.DMA((2,)),
        ],
        compiler_params=pltpu.CompilerParams(use_tc_tiling_on_sc=True,
                                             needs_layout_passes=False),
    )
    def k(gidx_hbm, pb_hbm, tab_hbm, out_hbm, idx_v, pb_v, rows_v, tb_v,
          isem, gsem, ssem):
        wid = lax.axis_index("s") * NC + lax.axis_index("c")
        f0 = wid * blk_per_w
        iota16 = jnp.arange(16, dtype=jnp.int32)

        def tb_of(f):
            return f // nbh, f % nbh

        def fire_idx(f, b):
            t, bh = tb_of(f)
            pltpu.async_copy(gidx_hbm.at[t, pl.ds(bh * BW, BW)],
                             idx_v.at[b], isem.at[b])
            pltpu.async_copy(pb_hbm.at[t, pl.ds(bh * BW, BW)],
                             pb_v.at[b], isem.at[b])

        def wait_idx(b):
            pltpu.make_async_copy(gidx_hbm.at[0, pl.ds(0, BW)],
                                  idx_v.at[b], isem.at[b]).wait()
            pltpu.make_async_copy(pb_hbm.at[0, pl.ds(0, BW)],
                                  pb_v.at[b], isem.at[b]).wait()

        def fire_gather(b):
            pltpu.async_copy(tab_hbm.at[idx_v.at[b]], rows_v.at[b],
                             gsem.at[b])

        def wait_gather(b):
            pltpu.make_async_copy(tab_hbm.at[pl.ds(0, BW)],
                                  rows_v.at[b], gsem.at[b]).wait()

        def fire_store(f, b):
            t, bh = tb_of(f)
            pltpu.async_copy(tb_v.at[b], out_hbm.at[t, :, bh], ssem.at[b])

        def wait_store(b):
            pltpu.make_async_copy(tb_v.at[b], out_hbm.at[0, :, 0],
                                  ssem.at[b]).wait()

        def transpose_scale(src_b, dst_b):
            pbs = [pb_v[src_b, pl.ds(kk * 16, 16)] for kk in range(8)]
            iotas = [iota16 + (kk * 16) for kk in range(8)]

            @plsc.parallel_loop(0, EMB, step=1, unroll=4)
            def _(e):
                ehi = e >> 3
                elo = e & 7
                for kk in range(8):
                    v = plsc.load_gather(rows_v.at[src_b],
                                         [iotas[kk], pbs[kk] + e])
                    tb_v[dst_b, ehi, elo, pl.ds(kk * 16, 16)] = v * SCALE

        for j in range(4):
            fire_idx(f0 + j, j)
        for j in range(3):
            wait_idx(j)
            fire_gather(j)

        def step(i, carry):
            for b4 in range(4):
                g = i * 4 + b4
                b2 = b4 % 2
                wait_gather(b4)
                transpose_scale(b4, b2)

                @pl.when(g >= 2)
                def _():
                    wait_store(b2)

                fire_store(f0 + g, b2)

                @pl.when(g + 3 < blk_per_w)
                def _():
                    wait_idx((g + 3) % 4)
                    fire_gather((g + 3) % 4)

                @pl.when(g + 4 < blk_per_w)
                def _():
                    fire_idx(f0 + g + 4, b4)

            return carry

        lax.fori_loop(0, blk_per_w // 4, step, 0)
        wait_store(0)
        wait_store(1)

    return k(gidx2d, pb2d, tab128)


def kernel(tokens, table):
    b0, b1 = tokens.shape
    tokT = jnp.swapaxes(tokens, 0, 1).astype(jnp.int32)
    gidx2d = tokT >> 1
    pb2d = (tokT & 1) << 6
    tab128 = jnp.reshape(table, (table.shape[0] // 2, 2 * EMB))
    outv = _sc_embed(gidx2d, pb2d, tab128)
    r = jnp.transpose(outv, (0, 1, 3, 2, 4))
    r = jnp.reshape(r, (b1, EMB, b0))
    return jnp.transpose(r, (2, 0, 1))


# speedup vs baseline: 2.3365x; 1.0062x over previous
import functools

import jax
import jax.numpy as jnp
from jax import lax
from jax.experimental import pallas as pl
from jax.experimental.pallas import tpu as pltpu
from jax.experimental.pallas import tpu_sc as plsc

EMB = 64
SCALE = 8.0

NC = 2
NS = 16
NW = NC * NS

BW = 128


def _sc_gather(gidx2d, tabR):
    t_dim, b_dim = gidx2d.shape
    nbh = b_dim // BW
    nblk = t_dim * nbh
    blk_per_w = nblk // NW

    mesh = plsc.VectorSubcoreMesh(core_axis_name="c", subcore_axis_name="s")

    @functools.partial(
        pl.kernel,
        mesh=mesh,
        out_type=jax.ShapeDtypeStruct((t_dim, 8, nbh, 8, BW), jnp.float32),
        scratch_types=[
            pltpu.VMEM((4, BW), jnp.int32),
            pltpu.VMEM((4, BW, EMB), jnp.float32),
            pltpu.VMEM((2, 8, 8, BW), jnp.float32),
            pltpu.SemaphoreType.DMA((4,)),
            pltpu.SemaphoreType.DMA((4,)),
            pltpu.SemaphoreType.DMA((2,)),
        ],
        compiler_params=pltpu.CompilerParams(use_tc_tiling_on_sc=False,
                                             needs_layout_passes=False),
    )
    def k(gidx_hbm, tab_hbm, out_hbm, idx_v, rows_v, tb_v, isem, gsem, ssem):
        wid = lax.axis_index("s") * NC + lax.axis_index("c")
        f0 = wid * blk_per_w
        iota16 = jnp.arange(16, dtype=jnp.int32)

        def tb_of(f):
            return f // nbh, f % nbh

        def fire_idx(f, b):
            t, bh = tb_of(f)
            pltpu.async_copy(gidx_hbm.at[t, pl.ds(bh * BW, BW)],
                             idx_v.at[b], isem.at[b])

        def wait_idx(b):
            pltpu.make_async_copy(gidx_hbm.at[0, pl.ds(0, BW)],
                                  idx_v.at[b], isem.at[b]).wait()

        def fire_gather(b):
            pltpu.async_copy(tab_hbm.at[idx_v.at[b]], rows_v.at[b],
                             gsem.at[b])

        def wait_gather(b):
            pltpu.make_async_copy(tab_hbm.at[pl.ds(0, BW)],
                                  rows_v.at[b], gsem.at[b]).wait()

        def fire_store(f, b):
            t, bh = tb_of(f)
            pltpu.async_copy(tb_v.at[b], out_hbm.at[t, :, bh], ssem.at[b])

        def wait_store(b):
            pltpu.make_async_copy(tb_v.at[b], out_hbm.at[0, :, 0],
                                  ssem.at[b]).wait()

        def transpose_scale(src_b, dst_b):
            iotas = [iota16 + (kk * 16) for kk in range(8)]

            @plsc.parallel_loop(0, EMB, step=1, unroll=4)
            def _(e):
                ehi = e >> 3
                elo = e & 7
                ev = iota16 * 0 + e
                for kk in range(8):
                    v = plsc.load_gather(rows_v.at[src_b], [iotas[kk], ev])
                    tb_v[dst_b, ehi, elo, pl.ds(kk * 16, 16)] = v * SCALE

        for j in range(4):
            fire_idx(f0 + j, j)
        for j in range(3):
            wait_idx(j)
            fire_gather(j)

        def step(i, carry):
            for b4 in range(4):
                g = i * 4 + b4
                b2 = b4 % 2
                wait_gather(b4)
                transpose_scale(b4, b2)

                @pl.when(g >= 2)
                def _():
                    wait_store(b2)

                fire_store(f0 + g, b2)

                @pl.when(g + 3 < blk_per_w)
                def _():
                    wait_idx((g + 3) % 4)
                    fire_gather((g + 3) % 4)

                @pl.when(g + 4 < blk_per_w)
                def _():
                    fire_idx(f0 + g + 4, b4)

            return carry

        lax.fori_loop(0, blk_per_w // 4, step, 0)
        wait_store(0)
        wait_store(1)

    return k(gidx2d, tabR)


def kernel(tokens, table):
    b0, b1 = tokens.shape
    tokT = jnp.swapaxes(tokens, 0, 1).astype(jnp.int32)
    outv = _sc_gather(tokT, table)
    r = jnp.transpose(outv, (0, 1, 3, 2, 4))
    r = jnp.reshape(r, (b1, EMB, b0))
    return jnp.transpose(r, (2, 0, 1))
